# baseline (speedup 1.0000x reference)
import functools
import math

import jax
import jax.numpy as jnp
from jax import lax
from jax.experimental import pallas as pl
from jax.experimental.pallas import tpu as pltpu
from jax.experimental.pallas import tpu_sc as plsc

_VOCAB = 1000000
_D = 64
_S = 200
_B = 4096

_NC = 2
_NS = 16
_NW = _NC * _NS
_BPW = _B // _NW
_QROWS = 32
_LANES = 16
_DSLICES = _D // _LANES
_G0 = 128
_G1 = _S - _G0
_OROWS = _S // 2


def _emb_body(x_hbm, emb_hbm, pos_hbm, out_hbm,
              blk_v, pos_v, in_v, out_v, gsem0, gsem1, ssem0, ssem1, bsem):
    wid = lax.axis_index("s") * _NC + lax.axis_index("c")
    b0 = wid * _BPW

    pltpu.sync_copy(x_hbm.at[pl.ds(b0, _QROWS)], blk_v.at[0])
    pltpu.sync_copy(pos_hbm, pos_v)
    pltpu.async_copy(x_hbm.at[pl.ds(b0 + _QROWS, _QROWS)], blk_v.at[1], bsem)

    gsems = (gsem0, gsem1)
    ssems = (ssem0, ssem1)

    def fire_gathers(r, ph):
        qph = (r // _QROWS) & 1
        rq = r % _QROWS
        pltpu.async_copy(emb_hbm.at[blk_v.at[qph, rq, pl.ds(0, _G0)]],
                         in_v.at[ph, pl.ds(0, _G0)], gsems[ph])
        pltpu.async_copy(emb_hbm.at[blk_v.at[qph, rq, pl.ds(_G0, _G1)]],
                         in_v.at[ph, pl.ds(_G0, _G1)], gsems[ph])

    def wait_gathers(ph):
        pltpu.make_async_copy(emb_hbm.at[blk_v.at[0, 0, pl.ds(0, _G0)]],
                              in_v.at[ph, pl.ds(0, _G0)], gsems[ph]).wait()
        pltpu.make_async_copy(emb_hbm.at[blk_v.at[0, 0, pl.ds(_G0, _G1)]],
                              in_v.at[ph, pl.ds(_G0, _G1)], gsems[ph]).wait()

    def row_pass(r, ph):
        wait_gathers(ph)

        @pl.when(r >= 2)
        def _():
            pltpu.make_async_copy(out_v.at[ph],
                                  out_hbm.at[pl.ds(b0 * _OROWS, _OROWS)],
                                  ssems[ph]).wait()

        def seq_step(t, carry):
            for u in range(2):
                jj = 2 * t + u
                for d in range(_DSLICES):
                    v = in_v[ph, jj, pl.ds(d * _LANES, _LANES)] * 8.0 \
                        + pos_v[jj, pl.ds(d * _LANES, _LANES)]
                    out_v[ph, t, pl.ds(u * _D + d * _LANES, _LANES)] = v
            return carry

        lax.fori_loop(0, _OROWS, seq_step, 0)
        pltpu.async_copy(out_v.at[ph],
                         out_hbm.at[pl.ds((b0 + r) * _OROWS, _OROWS)], ssems[ph])

        @pl.when(jnp.logical_and((r + 1) % _QROWS == 0,
                                 r + 1 + _QROWS < _BPW))
        def _():
            qn = (r + 1) // _QROWS + 1
            pltpu.async_copy(x_hbm.at[pl.ds(b0 + qn * _QROWS, _QROWS)],
                             blk_v.at[qn & 1], bsem)

        @pl.when(jnp.logical_and((r + 2) % _QROWS == 0, r + 2 < _BPW))
        def _():
            pltpu.make_async_copy(x_hbm.at[pl.ds(b0, _QROWS)],
                                  blk_v.at[0], bsem).wait()

        @pl.when(r + 2 < _BPW)
        def _():
            fire_gathers(r + 2, ph)

    fire_gathers(0, 0)
    fire_gathers(1, 1)

    def loop_body(i, carry):
        row_pass(2 * i, 0)
        row_pass(2 * i + 1, 1)
        return carry

    lax.fori_loop(0, _BPW // 2, loop_body, 0)
    pltpu.make_async_copy(out_v.at[0], out_hbm.at[pl.ds(0, _OROWS)], ssem0).wait()
    pltpu.make_async_copy(out_v.at[1], out_hbm.at[pl.ds(0, _OROWS)], ssem1).wait()


def kernel(x, emb_table, pos_table):
    emb_lin = emb_table.reshape(_VOCAB // 2, 2 * _D).reshape(_VOCAB, _D)

    mesh = plsc.VectorSubcoreMesh(core_axis_name="c", subcore_axis_name="s")
    run = functools.partial(
        pl.kernel,
        mesh=mesh,
        compiler_params=pltpu.CompilerParams(use_tc_tiling_on_sc=False,
                                             needs_layout_passes=True),
        out_type=jax.ShapeDtypeStruct((_B * _S // 2, 2 * _D), jnp.float32),
        scratch_types=[
            pltpu.VMEM((2, _QROWS, _S), jnp.int32),
            pltpu.VMEM((_S, _D), jnp.float32),
            pltpu.VMEM((2, _S, _D), jnp.float32),
            pltpu.VMEM((2, _OROWS, 2 * _D), jnp.float32),
            pltpu.SemaphoreType.DMA,
            pltpu.SemaphoreType.DMA,
            pltpu.SemaphoreType.DMA,
            pltpu.SemaphoreType.DMA,
            pltpu.SemaphoreType.DMA,
        ],
    )(_emb_body)
    out2 = run(x, emb_lin, pos_table)
    return out2.reshape(_B, _S, _D)
